# baseline (speedup 1.0000x reference)
import functools

import jax
import jax.numpy as jnp
from jax import lax
from jax.experimental import pallas as pl
from jax.experimental.pallas import tpu as pltpu
from jax.experimental.pallas import tpu_sc as plsc

_NC = 2
_NS = 16
_K = 128
_W = 128


def _round_up(a, b):
    return (a + b - 1) // b * b


@functools.cache
def _make_seg_sum(n_acc, cpr):
    mesh = plsc.VectorSubcoreMesh(
        core_axis_name="c", subcore_axis_name="s",
        num_cores=_NC, num_subcores=_NS)
    rt = n_acc // _NS
    cpt = _round_up(cpr + 4, 8)

    @functools.partial(
        pl.kernel,
        out_type=jax.ShapeDtypeStruct((_NC, n_acc, _W), jnp.float32),
        mesh=mesh,
        scratch_types=[
            pltpu.VMEM((4 * _K,), jnp.int32),
            pltpu.VMEM((cpt, _K), jnp.int32),
            pltpu.VMEM((2, _K, _W), jnp.float32),
            pltpu.VMEM_SHARED((n_acc, _W), jnp.float32),
            pltpu.SemaphoreType.DMA((4,)),
            pltpu.SemaphoreType.DMA((2,)),
            pltpu.SemaphoreType.DMA((2,)),
        ],
    )
    def seg_sum(feat_hbm, src_hbm, dst_hbm, zero_hbm, out_hbm,
                sidx_v, dst_v, rows_v, acc_sh, isem, gsem, ssem):
        c = lax.axis_index("c")
        s = lax.axis_index("s")
        wid = s * _NC + c
        row0 = s * rt
        base = wid * cpt * _K
        pltpu.sync_copy(dst_hbm.at[pl.ds(wid * cpt, cpt), :], dst_v)
        pltpu.sync_copy(zero_hbm.at[pl.ds(row0, rt)],
                        acc_sh.at[pl.ds(row0, rt)])
        plsc.subcore_barrier()

        def slot_ref(q):
            return sidx_v.at[pl.ds(q * _K, _K)]

        def issue_idx(chunk, q):
            pltpu.async_copy(
                src_hbm.at[pl.ds(base + chunk * _K, _K)],
                slot_ref(q), isem.at[q])

        def wait_idx(q):
            pltpu.make_async_copy(
                src_hbm.at[pl.ds(base, _K)], slot_ref(q), isem.at[q]).wait()

        def issue_gather(q, b):
            pltpu.async_copy(
                feat_hbm.at[slot_ref(q)], rows_v.at[b], gsem.at[b])

        def wait_gather(b):
            pltpu.make_async_copy(
                feat_hbm.at[slot_ref(0)], rows_v.at[b], gsem.at[b]).wait()

        def issue_scatter(j, b):
            pltpu.async_copy(
                rows_v.at[b], acc_sh.at[dst_v.at[j]], ssem.at[b], add=True)

        def wait_scatter(b):
            pltpu.make_async_copy(
                rows_v.at[b], acc_sh.at[dst_v.at[0]], ssem.at[b]).wait()

        for ch in range(3):
            issue_idx(ch, ch)
        wait_idx(0)
        issue_gather(0, 0)
        wait_gather(0)
        issue_scatter(jnp.int32(0), 0)
        issue_idx(jnp.int32(3), 3)
        wait_idx(1)
        issue_gather(1, 1)

        def visit(j, r):
            b = r % 2
            wait_gather(b)
            issue_scatter(j, b)
            issue_idx(j + 3, (r + 3) % 4)
            wait_scatter(1 - b)
            wait_idx((r + 1) % 4)
            issue_gather((r + 1) % 4, 1 - b)

        def body(m, carry):
            for r in range(1, 5):
                visit(m * 4 + r, r)
            return carry

        lax.fori_loop(0, cpr // 4, body, 0, unroll=False)

        wait_gather(1)
        wait_scatter(0)
        wait_idx(2)
        wait_idx(3)
        plsc.subcore_barrier()
        pltpu.sync_copy(acc_sh.at[pl.ds(row0, rt)],
                        out_hbm.at[c, pl.ds(row0, rt)])

    return seg_sum


def _mm_body(x_ref, w_ref, o_ref):
    o_ref[...] = jnp.dot(x_ref[...], w_ref[...],
                         preferred_element_type=jnp.float32)


def _mlp2_body(scale_ref, pre_ref, a0_ref, a1_ref, ba_ref, wb_ref, bb_ref,
               w2_ref, emb_ref, pre2_ref):
    t = (scale_ref[...] * pre_ref[...] + a0_ref[...] + a1_ref[...]
         + ba_ref[...])
    e = jnp.dot(jnp.maximum(t, 0.0), wb_ref[...],
                preferred_element_type=jnp.float32) + bb_ref[...]
    emb_ref[...] = e
    pre2_ref[...] = jnp.dot(jnp.maximum(e, 0.0), w2_ref[...],
                            preferred_element_type=jnp.float32)


def _mlp1_body(scale_ref, pre_ref, a0_ref, a1_ref, ba_ref, wb_ref, bb_ref,
               o_ref):
    t = (scale_ref[...] * pre_ref[...] + a0_ref[...] + a1_ref[...]
         + ba_ref[...])
    o_ref[...] = jnp.dot(jnp.maximum(t, 0.0), wb_ref[...],
                         preferred_element_type=jnp.float32) + bb_ref[...]


def _pad_cols(w, width):
    return jnp.concatenate(
        [w, jnp.zeros((w.shape[0], width - w.shape[1]), jnp.float32)], axis=1)


def _pad_rows(w, height):
    return jnp.concatenate(
        [w, jnp.zeros((height - w.shape[0], w.shape[1]), jnp.float32)])


def kernel(x, edge_index, W1a, b1a, W1b, b1b, eps1, W2a, b2a, W2b, b2b, eps2):
    n, nf = x.shape
    hc = W1a.shape[1]
    nc = W2b.shape[1]
    e = edge_index.shape[1]

    bk = 1024
    n_pad = _round_up(n + 1, bk)
    x_pad = jnp.concatenate([x, jnp.zeros((n_pad - n, nf), jnp.float32)])

    nw = _NC * _NS
    cpr = 4 * (-(-e // (nw * _K * 4)))
    e_pad = nw * _K * cpr
    src = edge_index[0]
    dst = edge_index[1]
    pad = e_pad - e
    if pad:
        src = jnp.concatenate([src, jnp.zeros((pad,), jnp.int32)])
        dst = jnp.concatenate([dst, jnp.full((pad,), n, jnp.int32)])
    cpt = _round_up(cpr + 4, 8)
    xtra = (cpt - cpr) * _K
    src = jnp.concatenate(
        [src.reshape(nw, cpr * _K),
         jnp.zeros((nw, xtra), jnp.int32)], axis=1).reshape(-1)
    dst = jnp.concatenate(
        [dst.reshape(nw, cpr * _K),
         jnp.full((nw, xtra), n, jnp.int32)], axis=1).reshape(-1, _K)

    zeros_acc = jnp.zeros((n_pad, _W), jnp.float32)
    seg_sum = _make_seg_sum(n_pad, cpr)

    grid = (n_pad // bk,)
    row_spec = lambda w: pl.BlockSpec((bk, w), lambda i: (i, 0))
    bcast_spec = lambda w: pl.BlockSpec((1, w), lambda i: (0, 0))

    pre1 = pl.pallas_call(
        _mm_body,
        grid=grid,
        in_specs=[row_spec(nf), pl.BlockSpec((nf, _W), lambda i: (0, 0))],
        out_specs=row_spec(_W),
        out_shape=jax.ShapeDtypeStruct((n_pad, _W), jnp.float32),
    )(x_pad, _pad_cols(W1a, _W))

    aggA = seg_sum(pre1, src, dst, zeros_acc)

    scale1 = jnp.full((1, _W), 1.0, jnp.float32) + eps1
    emb_pad, pre2 = pl.pallas_call(
        _mlp2_body,
        grid=grid,
        in_specs=[bcast_spec(_W), row_spec(_W), row_spec(_W), row_spec(_W),
                  bcast_spec(_W), pl.BlockSpec((_W, hc), lambda i: (0, 0)),
                  bcast_spec(hc), pl.BlockSpec((hc, _W), lambda i: (0, 0))],
        out_specs=[row_spec(hc), row_spec(_W)],
        out_shape=[jax.ShapeDtypeStruct((n_pad, hc), jnp.float32),
                   jax.ShapeDtypeStruct((n_pad, _W), jnp.float32)],
    )(scale1, pre1, aggA[0], aggA[1], _pad_cols(b1a.reshape(1, hc), _W),
      _pad_rows(W1b, _W), b1b.reshape(1, hc), _pad_cols(W2a, _W))

    aggB = seg_sum(pre2, src, dst, zeros_acc)

    scale2 = jnp.full((1, _W), 1.0, jnp.float32) + eps2
    logits_pad = pl.pallas_call(
        _mlp1_body,
        grid=grid,
        in_specs=[bcast_spec(_W), row_spec(_W), row_spec(_W), row_spec(_W),
                  bcast_spec(_W), pl.BlockSpec((_W, nc), lambda i: (0, 0)),
                  bcast_spec(nc)],
        out_specs=row_spec(nc),
        out_shape=jax.ShapeDtypeStruct((n_pad, nc), jnp.float32),
    )(scale2, pre2, aggB[0], aggB[1], _pad_cols(b2a.reshape(1, hc), _W),
      _pad_rows(W2b, _W), b2b.reshape(1, nc))

    return (logits_pad[:n], emb_pad[:n])


# speedup vs baseline: 2.0585x; 2.0585x over previous
import functools

import jax
import jax.numpy as jnp
from jax import lax
from jax.experimental import pallas as pl
from jax.experimental.pallas import tpu as pltpu
from jax.experimental.pallas import tpu_sc as plsc

_NC = 2
_NS = 16
_K = 128


def _round_up(a, b):
    return (a + b - 1) // b * b


@functools.cache
def _make_seg_sum(n_acc, feat, cpr):
    mesh = plsc.VectorSubcoreMesh(
        core_axis_name="c", subcore_axis_name="s",
        num_cores=_NC, num_subcores=_NS)
    rt = n_acc // _NS
    cpt = cpr + 2

    @functools.partial(
        pl.kernel,
        out_type=jax.ShapeDtypeStruct((_NC, n_acc, feat), jnp.float32),
        mesh=mesh,
        scratch_types=[
            pltpu.VMEM((2, _K), jnp.int32),
            pltpu.VMEM((2, _K), jnp.int32),
            pltpu.VMEM((2, _K, feat), jnp.float32),
            pltpu.VMEM_SHARED((n_acc, feat), jnp.float32),
            pltpu.SemaphoreType.DMA((2,)),
            pltpu.SemaphoreType.DMA((2,)),
            pltpu.SemaphoreType.DMA((2,)),
        ],
        compiler_params=pltpu.CompilerParams(use_tc_tiling_on_sc=False),
    )
    def seg_sum(feat_hbm, src_hbm, dst_hbm, zero_hbm, out_hbm,
                sidx_v, didx_v, rows_v, acc_sh, isem, dsem, gsem):
        c = lax.axis_index("c")
        s = lax.axis_index("s")
        wid = s * _NC + c
        row0 = s * rt
        base = wid * cpt * _K
        pltpu.sync_copy(zero_hbm.at[pl.ds(row0, rt)],
                        acc_sh.at[pl.ds(row0, rt)])
        plsc.subcore_barrier()

        def issue_idx(chunk, q):
            pltpu.async_copy(src_hbm.at[pl.ds(base + chunk * _K, _K)],
                             sidx_v.at[q], isem.at[q])
            pltpu.async_copy(dst_hbm.at[pl.ds(base + chunk * _K, _K)],
                             didx_v.at[q], dsem.at[q])

        def wait_idx(q):
            pltpu.make_async_copy(src_hbm.at[pl.ds(base, _K)],
                                  sidx_v.at[q], isem.at[q]).wait()
            pltpu.make_async_copy(dst_hbm.at[pl.ds(base, _K)],
                                  didx_v.at[q], dsem.at[q]).wait()

        def issue_gather(q, b):
            pltpu.async_copy(feat_hbm.at[sidx_v.at[q]],
                             rows_v.at[b], gsem.at[b])

        def wait_gather(b):
            pltpu.make_async_copy(feat_hbm.at[sidx_v.at[0]],
                                  rows_v.at[b], gsem.at[b]).wait()

        issue_idx(0, 0)
        issue_idx(jnp.int32(1), 1)
        wait_idx(0)
        issue_gather(0, 0)

        def visit(j, b):
            wait_gather(b)
            wait_idx(1 - b)
            issue_gather(1 - b, 1 - b)
            pltpu.sync_copy(rows_v.at[b], acc_sh.at[didx_v.at[b]], add=True)
            issue_idx(j + 2, b)

        def body(m, carry):
            visit(m * 2, 0)
            visit(m * 2 + 1, 1)
            return carry

        lax.fori_loop(0, cpr // 2, body, 0, unroll=False)

        wait_gather(0)
        wait_idx(1)
        plsc.subcore_barrier()
        pltpu.sync_copy(acc_sh.at[pl.ds(row0, rt)],
                        out_hbm.at[c, pl.ds(row0, rt)])

    return seg_sum


def _mm_body(x_ref, w_ref, o_ref):
    o_ref[...] = jnp.dot(x_ref[...], w_ref[...],
                         preferred_element_type=jnp.float32)


def _mlp2_body(scale_ref, pre_ref, a0_ref, a1_ref, ba_ref, wb_ref, bb_ref,
               w2_ref, emb_ref, pre2_ref):
    t = (scale_ref[...] * pre_ref[...] + a0_ref[...] + a1_ref[...]
         + ba_ref[...])
    e = jnp.dot(jnp.maximum(t, 0.0), wb_ref[...],
                preferred_element_type=jnp.float32) + bb_ref[...]
    emb_ref[...] = e
    pre2_ref[...] = jnp.dot(jnp.maximum(e, 0.0), w2_ref[...],
                            preferred_element_type=jnp.float32)


def _mlp1_body(scale_ref, pre_ref, a0_ref, a1_ref, ba_ref, wb_ref, bb_ref,
               o_ref):
    t = (scale_ref[...] * pre_ref[...] + a0_ref[...] + a1_ref[...]
         + ba_ref[...])
    o_ref[...] = jnp.dot(jnp.maximum(t, 0.0), wb_ref[...],
                         preferred_element_type=jnp.float32) + bb_ref[...]


def kernel(x, edge_index, W1a, b1a, W1b, b1b, eps1, W2a, b2a, W2b, b2b, eps2):
    n, nf = x.shape
    hc = W1a.shape[1]
    nc = W2b.shape[1]
    e = edge_index.shape[1]

    bk = 1024
    n_pad = _round_up(n + 1, bk)
    x_pad = jnp.concatenate([x, jnp.zeros((n_pad - n, nf), jnp.float32)])

    nw = _NC * _NS
    cpr = 2 * (-(-e // (nw * _K * 2)))
    e_pad = nw * _K * cpr
    src = edge_index[0]
    dst = edge_index[1]
    pad = e_pad - e
    if pad:
        src = jnp.concatenate([src, jnp.zeros((pad,), jnp.int32)])
        dst = jnp.concatenate([dst, jnp.full((pad,), n, jnp.int32)])
    cpt = cpr + 2
    xtra = (cpt - cpr) * _K
    src = jnp.concatenate(
        [src.reshape(nw, cpr * _K),
         jnp.zeros((nw, xtra), jnp.int32)], axis=1).reshape(-1)
    dst = jnp.concatenate(
        [dst.reshape(nw, cpr * _K),
         jnp.full((nw, xtra), n, jnp.int32)], axis=1).reshape(-1)

    zeros_acc = jnp.zeros((n_pad, hc), jnp.float32)
    seg_sum = _make_seg_sum(n_pad, hc, cpr)

    grid = (n_pad // bk,)
    row_spec = lambda w: pl.BlockSpec((bk, w), lambda i: (i, 0))
    bcast_spec = lambda w: pl.BlockSpec((1, w), lambda i: (0, 0))
    sq_spec = lambda w: pl.BlockSpec((w, w), lambda i: (0, 0))

    pre1 = pl.pallas_call(
        _mm_body,
        grid=grid,
        in_specs=[row_spec(nf), pl.BlockSpec((nf, hc), lambda i: (0, 0))],
        out_specs=row_spec(hc),
        out_shape=jax.ShapeDtypeStruct((n_pad, hc), jnp.float32),
    )(x_pad, W1a)

    aggA = seg_sum(pre1, src, dst, zeros_acc)

    scale1 = jnp.full((1, hc), 1.0, jnp.float32) + eps1
    emb_pad, pre2 = pl.pallas_call(
        _mlp2_body,
        grid=grid,
        in_specs=[bcast_spec(hc), row_spec(hc), row_spec(hc), row_spec(hc),
                  bcast_spec(hc), sq_spec(hc), bcast_spec(hc), sq_spec(hc)],
        out_specs=[row_spec(hc), row_spec(hc)],
        out_shape=[jax.ShapeDtypeStruct((n_pad, hc), jnp.float32),
                   jax.ShapeDtypeStruct((n_pad, hc), jnp.float32)],
    )(scale1, pre1, aggA[0], aggA[1], b1a.reshape(1, hc), W1b,
      b1b.reshape(1, hc), W2a)

    aggB = seg_sum(pre2, src, dst, zeros_acc)

    scale2 = jnp.full((1, hc), 1.0, jnp.float32) + eps2
    logits_pad = pl.pallas_call(
        _mlp1_body,
        grid=grid,
        in_specs=[bcast_spec(hc), row_spec(hc), row_spec(hc), row_spec(hc),
                  bcast_spec(hc), pl.BlockSpec((hc, nc), lambda i: (0, 0)),
                  bcast_spec(nc)],
        out_specs=row_spec(nc),
        out_shape=jax.ShapeDtypeStruct((n_pad, nc), jnp.float32),
    )(scale2, pre2, aggB[0], aggB[1], b2a.reshape(1, hc), W2b,
      b2b.reshape(1, nc))

    return (logits_pad[:n], emb_pad[:n])


# speedup vs baseline: 2.9489x; 1.4325x over previous
import functools

import jax
import jax.numpy as jnp
from jax import lax
from jax.experimental import pallas as pl
from jax.experimental.pallas import tpu as pltpu
from jax.experimental.pallas import tpu_sc as plsc

_NC = 2
_NS = 16
_K = 128


def _round_up(a, b):
    return (a + b - 1) // b * b


@functools.cache
def _make_seg_sum(n_acc, feat, cpw):
    mesh = plsc.VectorSubcoreMesh(
        core_axis_name="c", subcore_axis_name="s",
        num_cores=_NC, num_subcores=_NS)
    rt = n_acc // _NS

    @functools.partial(
        pl.kernel,
        out_type=jax.ShapeDtypeStruct((_NC, n_acc, feat), jnp.float32),
        mesh=mesh,
        scratch_types=[
            pltpu.VMEM((2, _K), jnp.int32),
            pltpu.VMEM((_K, feat), jnp.float32),
            pltpu.VMEM_SHARED((n_acc, feat), jnp.float32),
            pltpu.SemaphoreType.DMA,
        ],
        compiler_params=pltpu.CompilerParams(use_tc_tiling_on_sc=False),
    )
    def seg_sum(feat_hbm, idx_hbm, zero_hbm, out_hbm,
                idx_v, rows_v, acc_sh, gsem):
        c = lax.axis_index("c")
        s = lax.axis_index("s")
        wid = s * _NC + c
        row0 = s * rt
        base = wid * cpw * 2
        pltpu.sync_copy(zero_hbm.at[pl.ds(row0, rt)],
                        acc_sh.at[pl.ds(row0, rt)])
        plsc.subcore_barrier()

        def body(j, carry):
            pltpu.sync_copy(idx_hbm.at[pl.ds(base + j * 2, 2), :], idx_v)
            pltpu.async_copy(feat_hbm.at[idx_v.at[0]], rows_v, gsem).wait()
            pltpu.sync_copy(rows_v, acc_sh.at[idx_v.at[1]], add=True)
            return carry

        lax.fori_loop(0, cpw, body, 0)
        plsc.subcore_barrier()
        pltpu.sync_copy(acc_sh.at[pl.ds(row0, rt)],
                        out_hbm.at[c, pl.ds(row0, rt)])

    return seg_sum


def _mm_body(x_ref, w_ref, o_ref):
    o_ref[...] = jnp.dot(x_ref[...], w_ref[...],
                         preferred_element_type=jnp.float32)


def _mlp2_body(scale_ref, pre_ref, a0_ref, a1_ref, ba_ref, wb_ref, bb_ref,
               w2_ref, emb_ref, pre2_ref):
    t = (scale_ref[...] * pre_ref[...] + a0_ref[...] + a1_ref[...]
         + ba_ref[...])
    e = jnp.dot(jnp.maximum(t, 0.0), wb_ref[...],
                preferred_element_type=jnp.float32) + bb_ref[...]
    emb_ref[...] = e
    pre2_ref[...] = jnp.dot(jnp.maximum(e, 0.0), w2_ref[...],
                            preferred_element_type=jnp.float32)


def _mlp1_body(scale_ref, pre_ref, a0_ref, a1_ref, ba_ref, wb_ref, bb_ref,
               o_ref):
    t = (scale_ref[...] * pre_ref[...] + a0_ref[...] + a1_ref[...]
         + ba_ref[...])
    o_ref[...] = jnp.dot(jnp.maximum(t, 0.0), wb_ref[...],
                         preferred_element_type=jnp.float32) + bb_ref[...]


def kernel(x, edge_index, W1a, b1a, W1b, b1b, eps1, W2a, b2a, W2b, b2b, eps2):
    n, nf = x.shape
    hc = W1a.shape[1]
    nc = W2b.shape[1]
    e = edge_index.shape[1]

    bk = 1024
    n_pad = _round_up(n + 1, bk)
    x_pad = jnp.concatenate([x, jnp.zeros((n_pad - n, nf), jnp.float32)])

    nw = _NC * _NS
    cpw = -(-e // (nw * _K))
    e_pad = nw * _K * cpw
    src = edge_index[0]
    dst = edge_index[1]
    pad = e_pad - e
    if pad:
        src = jnp.concatenate([src, jnp.zeros((pad,), jnp.int32)])
        dst = jnp.concatenate([dst, jnp.full((pad,), n, jnp.int32)])
    idx = jnp.stack(
        [src.reshape(-1, _K), dst.reshape(-1, _K)], axis=1).reshape(-1, _K)

    zeros_acc = jnp.zeros((n_pad, hc), jnp.float32)
    seg_sum = _make_seg_sum(n_pad, hc, cpw)

    grid = (n_pad // bk,)
    row_spec = lambda w: pl.BlockSpec((bk, w), lambda i: (i, 0))
    bcast_spec = lambda w: pl.BlockSpec((1, w), lambda i: (0, 0))
    sq_spec = lambda w: pl.BlockSpec((w, w), lambda i: (0, 0))

    pre1 = pl.pallas_call(
        _mm_body,
        grid=grid,
        in_specs=[row_spec(nf), pl.BlockSpec((nf, hc), lambda i: (0, 0))],
        out_specs=row_spec(hc),
        out_shape=jax.ShapeDtypeStruct((n_pad, hc), jnp.float32),
    )(x_pad, W1a)

    aggA = seg_sum(pre1, idx, zeros_acc)

    scale1 = jnp.full((1, hc), 1.0, jnp.float32) + eps1
    emb_pad, pre2 = pl.pallas_call(
        _mlp2_body,
        grid=grid,
        in_specs=[bcast_spec(hc), row_spec(hc), row_spec(hc), row_spec(hc),
                  bcast_spec(hc), sq_spec(hc), bcast_spec(hc), sq_spec(hc)],
        out_specs=[row_spec(hc), row_spec(hc)],
        out_shape=[jax.ShapeDtypeStruct((n_pad, hc), jnp.float32),
                   jax.ShapeDtypeStruct((n_pad, hc), jnp.float32)],
    )(scale1, pre1, aggA[0], aggA[1], b1a.reshape(1, hc), W1b,
      b1b.reshape(1, hc), W2a)

    aggB = seg_sum(pre2, idx, zeros_acc)

    scale2 = jnp.full((1, hc), 1.0, jnp.float32) + eps2
    logits_pad = pl.pallas_call(
        _mlp1_body,
        grid=grid,
        in_specs=[bcast_spec(hc), row_spec(hc), row_spec(hc), row_spec(hc),
                  bcast_spec(hc), pl.BlockSpec((hc, nc), lambda i: (0, 0)),
                  bcast_spec(nc)],
        out_specs=row_spec(nc),
        out_shape=jax.ShapeDtypeStruct((n_pad, nc), jnp.float32),
    )(scale2, pre2, aggB[0], aggB[1], b2a.reshape(1, hc), W2b,
      b2b.reshape(1, nc))

    return (logits_pad[:n], emb_pad[:n])


# speedup vs baseline: 3.3355x; 1.1311x over previous
import functools

import jax
import jax.numpy as jnp
from jax import lax
from jax.experimental import pallas as pl
from jax.experimental.pallas import tpu as pltpu
from jax.experimental.pallas import tpu_sc as plsc

_NC = 2
_NS = 16
_K = 128


def _round_up(a, b):
    return (a + b - 1) // b * b


@functools.cache
def _make_seg_sum(n_acc, feat, cpw):
    mesh = plsc.VectorSubcoreMesh(
        core_axis_name="c", subcore_axis_name="s",
        num_cores=_NC, num_subcores=_NS)
    rt = n_acc // _NS

    @functools.partial(
        pl.kernel,
        out_type=jax.ShapeDtypeStruct((_NC, n_acc, feat), jnp.float32),
        mesh=mesh,
        scratch_types=[
            pltpu.VMEM((2 * cpw, _K), jnp.int32),
            pltpu.VMEM((_K, feat), jnp.float32),
            pltpu.VMEM_SHARED((n_acc, feat), jnp.float32),
            pltpu.SemaphoreType.DMA,
        ],
        compiler_params=pltpu.CompilerParams(use_tc_tiling_on_sc=False),
    )
    def seg_sum(feat_hbm, idx_hbm, zero_hbm, out_hbm,
                idx_v, rows_v, acc_sh, gsem):
        c = lax.axis_index("c")
        s = lax.axis_index("s")
        wid = s * _NC + c
        row0 = s * rt
        base = wid * cpw * 2
        pltpu.sync_copy(idx_hbm.at[pl.ds(base, 2 * cpw), :], idx_v)
        pltpu.sync_copy(zero_hbm.at[pl.ds(row0, rt)],
                        acc_sh.at[pl.ds(row0, rt)])
        plsc.subcore_barrier()

        def body(j, carry):
            pltpu.async_copy(
                feat_hbm.at[idx_v.at[2 * j]], rows_v, gsem).wait()
            pltpu.sync_copy(rows_v, acc_sh.at[idx_v.at[2 * j + 1]], add=True)
            return carry

        lax.fori_loop(0, cpw, body, 0)
        plsc.subcore_barrier()
        pltpu.sync_copy(acc_sh.at[pl.ds(row0, rt)],
                        out_hbm.at[c, pl.ds(row0, rt)])

    return seg_sum


def _mm_body(x_ref, w_ref, o_ref):
    o_ref[...] = jnp.dot(x_ref[...], w_ref[...],
                         preferred_element_type=jnp.float32)


def _mlp2_body(scale_ref, pre_ref, a0_ref, a1_ref, ba_ref, wb_ref, bb_ref,
               w2_ref, emb_ref, pre2_ref):
    t = (scale_ref[...] * pre_ref[...] + a0_ref[...] + a1_ref[...]
         + ba_ref[...])
    e = jnp.dot(jnp.maximum(t, 0.0), wb_ref[...],
                preferred_element_type=jnp.float32) + bb_ref[...]
    emb_ref[...] = e
    pre2_ref[...] = jnp.dot(jnp.maximum(e, 0.0), w2_ref[...],
                            preferred_element_type=jnp.float32)


def _mlp1_body(scale_ref, pre_ref, a0_ref, a1_ref, ba_ref, wb_ref, bb_ref,
               o_ref):
    t = (scale_ref[...] * pre_ref[...] + a0_ref[...] + a1_ref[...]
         + ba_ref[...])
    o_ref[...] = jnp.dot(jnp.maximum(t, 0.0), wb_ref[...],
                         preferred_element_type=jnp.float32) + bb_ref[...]


def kernel(x, edge_index, W1a, b1a, W1b, b1b, eps1, W2a, b2a, W2b, b2b, eps2):
    n, nf = x.shape
    hc = W1a.shape[1]
    nc = W2b.shape[1]
    e = edge_index.shape[1]

    bk = 1024
    n_pad = _round_up(n + 1, bk)
    x_pad = jnp.concatenate([x, jnp.zeros((n_pad - n, nf), jnp.float32)])

    nw = _NC * _NS
    cpw = -(-e // (nw * _K))
    e_pad = nw * _K * cpw
    src = edge_index[0]
    dst = edge_index[1]
    pad = e_pad - e
    if pad:
        src = jnp.concatenate([src, jnp.zeros((pad,), jnp.int32)])
        dst = jnp.concatenate([dst, jnp.full((pad,), n, jnp.int32)])
    idx = jnp.stack(
        [src.reshape(-1, _K), dst.reshape(-1, _K)], axis=1).reshape(-1, _K)

    zeros_acc = jnp.zeros((n_pad, hc), jnp.float32)
    seg_sum = _make_seg_sum(n_pad, hc, cpw)

    grid = (n_pad // bk,)
    row_spec = lambda w: pl.BlockSpec((bk, w), lambda i: (i, 0))
    bcast_spec = lambda w: pl.BlockSpec((1, w), lambda i: (0, 0))
    sq_spec = lambda w: pl.BlockSpec((w, w), lambda i: (0, 0))

    pre1 = pl.pallas_call(
        _mm_body,
        grid=grid,
        in_specs=[row_spec(nf), pl.BlockSpec((nf, hc), lambda i: (0, 0))],
        out_specs=row_spec(hc),
        out_shape=jax.ShapeDtypeStruct((n_pad, hc), jnp.float32),
    )(x_pad, W1a)

    aggA = seg_sum(pre1, idx, zeros_acc)

    scale1 = jnp.full((1, hc), 1.0, jnp.float32) + eps1
    emb_pad, pre2 = pl.pallas_call(
        _mlp2_body,
        grid=grid,
        in_specs=[bcast_spec(hc), row_spec(hc), row_spec(hc), row_spec(hc),
                  bcast_spec(hc), sq_spec(hc), bcast_spec(hc), sq_spec(hc)],
        out_specs=[row_spec(hc), row_spec(hc)],
        out_shape=[jax.ShapeDtypeStruct((n_pad, hc), jnp.float32),
                   jax.ShapeDtypeStruct((n_pad, hc), jnp.float32)],
    )(scale1, pre1, aggA[0], aggA[1], b1a.reshape(1, hc), W1b,
      b1b.reshape(1, hc), W2a)

    aggB = seg_sum(pre2, idx, zeros_acc)

    scale2 = jnp.full((1, hc), 1.0, jnp.float32) + eps2
    logits_pad = pl.pallas_call(
        _mlp1_body,
        grid=grid,
        in_specs=[bcast_spec(hc), row_spec(hc), row_spec(hc), row_spec(hc),
                  bcast_spec(hc), pl.BlockSpec((hc, nc), lambda i: (0, 0)),
                  bcast_spec(nc)],
        out_specs=row_spec(nc),
        out_shape=jax.ShapeDtypeStruct((n_pad, nc), jnp.float32),
    )(scale2, pre2, aggB[0], aggB[1], b2a.reshape(1, hc), W2b,
      b2b.reshape(1, nc))

    return (logits_pad[:n], emb_pad[:n])
